# baseline (speedup 1.0000x reference)
import functools

import jax
import jax.numpy as jnp
from jax.experimental import pallas as pl
import numpy as np


def _pe_kernel(q_ref, d_ref, wt_ref, o_ref, *, n_heads: int, n_j: int, bi: int):
    _HI = np.uint32(0xFFFF0000)
    idx = jnp.minimum(d_ref[...], 128)
    idxc = jnp.minimum(idx, 127)
    mask = idx >= 128
    qs = q_ref[...].reshape(n_heads * bi, q_ref.shape[-1])
    qdr = jnp.dot(qs, wt_ref[...],
                  preferred_element_type=jnp.float32)
    bits = jax.lax.bitcast_convert_type(qdr, jnp.uint32)
    rb = (bits + np.uint32(0x8000)) & _HI
    packed = [rb[(2 * hp) * bi:(2 * hp + 1) * bi]
              | (rb[(2 * hp + 1) * bi:(2 * hp + 2) * bi] >> 16)
              for hp in range(n_heads // 2)]
    for jc in range(n_j):
        sl = slice(jc * 128, (jc + 1) * 128)
        ic = idxc[:, sl]
        m = mask[:, sl]
        for hp in range(n_heads // 2):
            g = jnp.take_along_axis(packed[hp][:, :128], ic, axis=-1)
            s = jnp.where(m, packed[hp][:, 128:129], g)
            o_ref[2 * hp, :, sl] = jax.lax.bitcast_convert_type(
                s & _HI, jnp.float32)
            o_ref[2 * hp + 1, :, sl] = jax.lax.bitcast_convert_type(
                s << 16, jnp.float32)


def kernel(q, dist_matrices, W):
    B, H, S, DK = q.shape
    P = W.shape[0]
    assert B == 1 and P <= 129 and H % 2 == 0
    Wt = jnp.zeros((DK, 256), dtype=W.dtype).at[:, :P].set(W.T)
    q2 = q[0]
    dist = dist_matrices[0]
    BI = 128
    body = functools.partial(_pe_kernel, n_heads=H, n_j=S // 128, bi=BI)
    out = pl.pallas_call(
        body,
        grid=(S // BI,),
        in_specs=[
            pl.BlockSpec((H, BI, DK), lambda i: (0, i, 0)),
            pl.BlockSpec((BI, S), lambda i: (i, 0)),
            pl.BlockSpec((DK, 256), lambda i: (0, 0)),
        ],
        out_specs=pl.BlockSpec((H, BI, S), lambda i: (0, i, 0)),
        out_shape=jax.ShapeDtypeStruct((H, S, S), jnp.float32),
    )(q2, dist, Wt)
    return out[None]
